# baseline (speedup 1.0000x reference)
import functools

import jax
import jax.numpy as jnp
from jax import lax
from jax.experimental import pallas as pl
from jax.experimental.pallas import tpu as pltpu
from jax.experimental.pallas import tpu_sc as plsc

E = 16
N = 8192
D = 2048
CAP = 2 * N // E

_NC = 2
_NS = 16
NW = _NC * _NS
CHUNK = 16
NCHUNK = E * CAP // (NW * CHUNK)


def _sc_body(in_hbm, srcp_hbm, cnt_hbm, z_hbm, out_hbm,
             idx_all_v, cnts_v, zrow_v, rows0_v, rows1_v,
             gsem, ssem0, ssem1):
    wid = lax.axis_index("s") * _NC + lax.axis_index("c")

    pltpu.sync_copy(srcp_hbm.at[wid], idx_all_v)
    pltpu.sync_copy(cnt_hbm.at[wid], cnts_v)
    pltpu.sync_copy(z_hbm, zrow_v)
    c_lo = cnts_v[pl.ds(0, 16)]
    c_hi = cnts_v[pl.ds(16, 16)]

    rows_v = (rows0_v, rows1_v)
    ssem = (ssem0, ssem1)

    for k in range(NCHUNK):
        b = k % 2
        nv = (c_lo if k < 16 else c_hi)[k % 16]
        c0 = wid * CHUNK + k * (NW * CHUNK)

        if k >= 2:
            pltpu.make_async_copy(z_hbm, rows_v[b], ssem[b]).wait()

        @pl.when(nv > 0)
        def _(b=b, k=k, nv=nv, c0=c0):
            pltpu.async_copy(in_hbm.at[idx_all_v.at[k]], rows_v[b], gsem
                             ).wait()

            @pl.when(nv < CHUNK)
            def _():
                @pl.loop(nv, CHUNK)
                def _(r):
                    @pl.loop(0, D // 16)
                    def _(j):
                        rows_v[b][r, pl.ds(j * 16, 16)] = jnp.zeros(
                            (16,), jnp.float32)

            pltpu.async_copy(rows_v[b], out_hbm.at[pl.ds(c0, CHUNK)], ssem[b])

        @pl.when(nv == 0)
        def _(b=b, c0=c0):
            pltpu.async_copy(zrow_v, out_hbm.at[pl.ds(c0, CHUNK)], ssem[b])

    pltpu.make_async_copy(z_hbm, rows0_v, ssem0).wait()
    pltpu.make_async_copy(z_hbm, rows1_v, ssem1).wait()


@jax.jit
def _route_gather(inputs, srcp, cnts, zrows):
    k = pl.kernel(
        _sc_body,
        out_type=jax.ShapeDtypeStruct((E * CAP, D), jnp.float32),
        mesh=plsc.VectorSubcoreMesh(core_axis_name="c", subcore_axis_name="s"),
        scratch_types=[
            pltpu.VMEM((NCHUNK, CHUNK), jnp.int32),
            pltpu.VMEM((NCHUNK,), jnp.int32),
            pltpu.VMEM((CHUNK, D), jnp.float32),
            pltpu.VMEM((CHUNK, D), jnp.float32),
            pltpu.VMEM((CHUNK, D), jnp.float32),
            pltpu.SemaphoreType.DMA,
            pltpu.SemaphoreType.DMA,
            pltpu.SemaphoreType.DMA,
        ],
    )
    return k(inputs, srcp, cnts, zrows)


_ROUTING = None


def _routing_tables():
    global _ROUTING
    if _ROUTING is None:
        score = jax.random.normal(jax.random.key(42), (N, E),
                                  dtype=jnp.float32)
        _, top_idx = jax.lax.top_k(score, 1)
        dst = top_idx[:, 0]
        onehot = (dst[:, None] == jnp.arange(E)[None, :]).astype(jnp.int32)
        pos = jnp.cumsum(onehot, axis=0) - 1
        pos_in_expert = jnp.take_along_axis(pos, dst[:, None], axis=1)[:, 0]
        counts = jnp.sum(onehot, axis=0)
        filled = jnp.minimum(counts, CAP)
        slot = jnp.where(pos_in_expert < CAP, dst * CAP + pos_in_expert,
                         E * CAP)
        src = jnp.zeros((E * CAP,), jnp.int32).at[slot].set(
            jnp.arange(N, dtype=jnp.int32), mode="drop")
        srcp = src.reshape(NCHUNK, NW, CHUNK).transpose(1, 0, 2)
        g = jnp.arange(E * CAP // CHUNK)
        cnt_chunk = jnp.clip(filled[g // (CAP // CHUNK)]
                             - (g % (CAP // CHUNK)) * CHUNK, 0, CHUNK)
        cnts = cnt_chunk.reshape(NCHUNK, NW).T.astype(jnp.int32)
        zrows = jnp.zeros((CHUNK, D), jnp.float32)
        _ROUTING = (jax.device_get(srcp), jax.device_get(cnts),
                    jax.device_get(zrows))
    return _ROUTING


def kernel(inputs):
    srcp, cnts, zrows = _routing_tables()
    out = _route_gather(inputs, jnp.asarray(srcp), jnp.asarray(cnts),
                        jnp.asarray(zrows))
    return out.reshape(E, CAP, inputs.shape[1])


# speedup vs baseline: 1.7571x; 1.7571x over previous
import functools

import jax
import jax.numpy as jnp
from jax import lax
from jax.experimental import pallas as pl
from jax.experimental.pallas import tpu as pltpu
from jax.experimental.pallas import tpu_sc as plsc

E = 16
N = 8192
D = 2048
CAP = 2 * N // E

_NC = 2
_NS = 16
NW = _NC * _NS
CHUNK = 16
NCHUNK = E * CAP // (NW * CHUNK)


def _sc_body(in_hbm, srcp_hbm, cnt_hbm, z_hbm, out_hbm,
             idx_all_v, cnts_v, zrow_v, rows0_v, rows1_v,
             gsem, ssem0, ssem1):
    wid = lax.axis_index("s") * _NC + lax.axis_index("c")

    pltpu.sync_copy(srcp_hbm.at[wid], idx_all_v)
    pltpu.sync_copy(cnt_hbm.at[wid], cnts_v)
    pltpu.sync_copy(z_hbm, zrow_v)
    c_lo = cnts_v[pl.ds(0, 16)]
    c_hi = cnts_v[pl.ds(16, 16)]

    rows_v = (rows0_v, rows1_v)
    ssem = (ssem0, ssem1)

    for k in range(NCHUNK):
        b = k % 2
        nv = (c_lo if k < 16 else c_hi)[k % 16]
        c0 = wid * CHUNK + k * (NW * CHUNK)

        if k >= 2:
            pltpu.make_async_copy(z_hbm, rows_v[b], ssem[b]).wait()

        @pl.when(nv > 0)
        def _(b=b, k=k, nv=nv, c0=c0):
            pltpu.async_copy(in_hbm.at[idx_all_v.at[k]], rows_v[b], gsem
                             ).wait()

            @pl.when(nv < CHUNK)
            def _():
                @pl.loop(nv, CHUNK)
                def _(r):
                    @pl.loop(0, D // 16)
                    def _(j):
                        rows_v[b][r, pl.ds(j * 16, 16)] = jnp.zeros(
                            (16,), jnp.float32)

            pltpu.async_copy(rows_v[b], out_hbm.at[pl.ds(c0, CHUNK)], ssem[b])

        @pl.when(nv == 0)
        def _(b=b, c0=c0):
            pltpu.async_copy(zrow_v, out_hbm.at[pl.ds(c0, CHUNK)], ssem[b])

    pltpu.make_async_copy(z_hbm, rows0_v, ssem0).wait()
    pltpu.make_async_copy(z_hbm, rows1_v, ssem1).wait()


@jax.jit
def _route_gather(inputs, srcp, cnts, zrows):
    k = pl.kernel(
        _sc_body,
        out_type=jax.ShapeDtypeStruct((E * CAP, D), jnp.float32),
        mesh=plsc.VectorSubcoreMesh(core_axis_name="c", subcore_axis_name="s"),
        scratch_types=[
            pltpu.VMEM((NCHUNK, CHUNK), jnp.int32),
            pltpu.VMEM((NCHUNK,), jnp.int32),
            pltpu.VMEM((CHUNK, D), jnp.float32),
            pltpu.VMEM((CHUNK, D), jnp.float32),
            pltpu.VMEM((CHUNK, D), jnp.float32),
            pltpu.SemaphoreType.DMA,
            pltpu.SemaphoreType.DMA,
            pltpu.SemaphoreType.DMA,
        ],
    )
    return k(inputs, srcp, cnts, zrows)


_ROUTING = None


def _routing_tables():
    global _ROUTING
    if _ROUTING is None:
      with jax.ensure_compile_time_eval():
        score = jax.random.normal(jax.random.key(42), (N, E),
                                  dtype=jnp.float32)
        _, top_idx = jax.lax.top_k(score, 1)
        dst = top_idx[:, 0]
        onehot = (dst[:, None] == jnp.arange(E)[None, :]).astype(jnp.int32)
        pos = jnp.cumsum(onehot, axis=0) - 1
        pos_in_expert = jnp.take_along_axis(pos, dst[:, None], axis=1)[:, 0]
        counts = jnp.sum(onehot, axis=0)
        filled = jnp.minimum(counts, CAP)
        slot = jnp.where(pos_in_expert < CAP, dst * CAP + pos_in_expert,
                         E * CAP)
        src = jnp.zeros((E * CAP,), jnp.int32).at[slot].set(
            jnp.arange(N, dtype=jnp.int32), mode="drop")
        srcp = src.reshape(NCHUNK, NW, CHUNK).transpose(1, 0, 2)
        g = jnp.arange(E * CAP // CHUNK)
        cnt_chunk = jnp.clip(filled[g // (CAP // CHUNK)]
                             - (g % (CAP // CHUNK)) * CHUNK, 0, CHUNK)
        cnts = cnt_chunk.reshape(NCHUNK, NW).T.astype(jnp.int32)
        zrows = jnp.zeros((CHUNK, D), jnp.float32)
        _ROUTING = (jax.device_get(srcp), jax.device_get(cnts),
                    jax.device_get(zrows))
    return _ROUTING


def kernel(inputs):
    srcp, cnts, zrows = _routing_tables()
    out = _route_gather(inputs, jnp.asarray(srcp), jnp.asarray(cnts),
                        jnp.asarray(zrows))
    return out.reshape(E, CAP, inputs.shape[1])


# speedup vs baseline: 1.9616x; 1.1164x over previous
import functools

import jax
import jax.numpy as jnp
from jax import lax
from jax.experimental import pallas as pl
from jax.experimental.pallas import tpu as pltpu
from jax.experimental.pallas import tpu_sc as plsc

E = 16
N = 8192
D = 2048
CAP = 2 * N // E

_NC = 2
_NS = 16
NW = _NC * _NS
CHUNK = 16
ZROWS = 8
NCHUNK = E * CAP // (NW * CHUNK)


def _sc_body(in_hbm, srcp_hbm, cnt_hbm, z_hbm, out_hbm,
             idx_all_v, cnts_v, zrow_v, rows0_v, rows1_v, rows2_v,
             gsem0, gsem1, gsem2, ssem0, ssem1, ssem2):
    wid = lax.axis_index("s") * _NC + lax.axis_index("c")

    pltpu.sync_copy(srcp_hbm.at[wid], idx_all_v)
    pltpu.sync_copy(cnt_hbm.at[wid], cnts_v)
    pltpu.sync_copy(z_hbm, zrow_v)
    c_lo = cnts_v[pl.ds(0, 16)]
    c_hi = cnts_v[pl.ds(16, 16)]

    rows_v = (rows0_v, rows1_v, rows2_v)
    gsem = (gsem0, gsem1, gsem2)
    ssem = (ssem0, ssem1, ssem2)

    def nv_of(k):
        return (c_lo if k < 16 else c_hi)[k % 16]

    def issue_gather(k):
        @pl.when(nv_of(k) > 0)
        def _():
            pltpu.async_copy(in_hbm.at[idx_all_v.at[k]], rows_v[k % 3],
                             gsem[k % 3])

    issue_gather(0)
    issue_gather(1)

    for k in range(NCHUNK):
        b = k % 3
        nv = nv_of(k)
        c0 = wid * CHUNK + k * (NW * CHUNK)

        @pl.when(nv > 0)
        def _(b=b, k=k, nv=nv, c0=c0):
            pltpu.make_async_copy(in_hbm.at[pl.ds(0, CHUNK)], rows_v[b],
                                  gsem[b]).wait()

            @pl.when(nv < CHUNK)
            def _():
                @pl.loop(nv, CHUNK)
                def _(r):
                    @pl.loop(0, D // 16)
                    def _(j):
                        rows_v[b][r, pl.ds(j * 16, 16)] = jnp.zeros(
                            (16,), jnp.float32)

            pltpu.async_copy(rows_v[b], out_hbm.at[pl.ds(c0, CHUNK)], ssem[b])

        @pl.when(nv == 0)
        def _(b=b, c0=c0):
            pltpu.async_copy(zrow_v, out_hbm.at[pl.ds(c0, ZROWS)], ssem[b])
            pltpu.async_copy(zrow_v, out_hbm.at[pl.ds(c0 + ZROWS, ZROWS)],
                             ssem[b])

        j = k + 2
        if j < NCHUNK:
            if j >= 3:
                pltpu.make_async_copy(in_hbm.at[pl.ds(0, CHUNK)],
                                      rows_v[j % 3], ssem[j % 3]).wait()
            issue_gather(j)

    for b in range(3):
        pltpu.make_async_copy(in_hbm.at[pl.ds(0, CHUNK)], rows_v[b],
                              ssem[b]).wait()


@jax.jit
def _route_gather(inputs, srcp, cnts, zrows):
    k = pl.kernel(
        _sc_body,
        out_type=jax.ShapeDtypeStruct((E * CAP, D), jnp.float32),
        mesh=plsc.VectorSubcoreMesh(core_axis_name="c", subcore_axis_name="s"),
        scratch_types=[
            pltpu.VMEM((NCHUNK, CHUNK), jnp.int32),
            pltpu.VMEM((NCHUNK,), jnp.int32),
            pltpu.VMEM((ZROWS, D), jnp.float32),
            pltpu.VMEM((CHUNK, D), jnp.float32),
            pltpu.VMEM((CHUNK, D), jnp.float32),
            pltpu.VMEM((CHUNK, D), jnp.float32),
            pltpu.SemaphoreType.DMA,
            pltpu.SemaphoreType.DMA,
            pltpu.SemaphoreType.DMA,
            pltpu.SemaphoreType.DMA,
            pltpu.SemaphoreType.DMA,
            pltpu.SemaphoreType.DMA,
        ],
    )
    return k(inputs, srcp, cnts, zrows)


_ROUTING = None


def _routing_tables():
    global _ROUTING
    if _ROUTING is None:
      with jax.ensure_compile_time_eval():
        score = jax.random.normal(jax.random.key(42), (N, E),
                                  dtype=jnp.float32)
        _, top_idx = jax.lax.top_k(score, 1)
        dst = top_idx[:, 0]
        onehot = (dst[:, None] == jnp.arange(E)[None, :]).astype(jnp.int32)
        pos = jnp.cumsum(onehot, axis=0) - 1
        pos_in_expert = jnp.take_along_axis(pos, dst[:, None], axis=1)[:, 0]
        counts = jnp.sum(onehot, axis=0)
        filled = jnp.minimum(counts, CAP)
        slot = jnp.where(pos_in_expert < CAP, dst * CAP + pos_in_expert,
                         E * CAP)
        src = jnp.zeros((E * CAP,), jnp.int32).at[slot].set(
            jnp.arange(N, dtype=jnp.int32), mode="drop")
        srcp = src.reshape(NCHUNK, NW, CHUNK).transpose(1, 0, 2)
        g = jnp.arange(E * CAP // CHUNK)
        cnt_chunk = jnp.clip(filled[g // (CAP // CHUNK)]
                             - (g % (CAP // CHUNK)) * CHUNK, 0, CHUNK)
        cnts = cnt_chunk.reshape(NCHUNK, NW).T.astype(jnp.int32)
        zrows = jnp.zeros((ZROWS, D), jnp.float32)
        _ROUTING = (jax.device_get(srcp), jax.device_get(cnts),
                    jax.device_get(zrows))
    return _ROUTING


def kernel(inputs):
    srcp, cnts, zrows = _routing_tables()
    out = _route_gather(inputs, jnp.asarray(srcp), jnp.asarray(cnts),
                        jnp.asarray(zrows))
    return out.reshape(E, CAP, inputs.shape[1])
